# Initial kernel scaffold; baseline (speedup 1.0000x reference)
import jax
import jax.numpy as jnp
from jax.experimental import pallas as pl


def kernel(x, edge_index, W_self1, W_neigh1, b1, W_self2, W_neigh2, b2):
    raise NotImplementedError("write your pallas kernel here")


# speedup vs baseline: 6.3318x; 6.3318x over previous
import functools

import jax
import jax.numpy as jnp
from jax import lax
from jax.experimental import pallas as pl
from jax.experimental.pallas import tpu as pltpu
from jax.experimental.pallas import tpu_sc as plsc

N = 10000
NP = 10240
E = 320000
D = 128
W_AUG = 144
CHUNK = 128
NCHUNKS = E // CHUNK
NTILES = 32
ROWS_PER_TILE = NP // 16


def _make_sc_agg(w):
    mesh = plsc.VectorSubcoreMesh(core_axis_name="c", subcore_axis_name="s",
                                  num_cores=2, num_subcores=16)

    @functools.partial(
        pl.kernel,
        out_type=jax.ShapeDtypeStruct((2 * NP, w), jnp.float32),
        mesh=mesh,
        scratch_types=[
            pltpu.VMEM((CHUNK,), jnp.int32),
            pltpu.VMEM((CHUNK,), jnp.int32),
            pltpu.VMEM((CHUNK, w), jnp.float32),
            pltpu.VMEM_SHARED((NP, w), jnp.float32),
            pltpu.SemaphoreType.DMA,
        ],
        compiler_params=pltpu.CompilerParams(use_tc_tiling_on_sc=False),
    )
    def sc_agg(x_hbm, src_hbm, dst_hbm, zeros_hbm, out_hbm,
               src_v, dst_v, rows_v, acc_sh, sem):
        c = lax.axis_index("c")
        s = lax.axis_index("s")
        wid = c * 16 + s

        pltpu.sync_copy(zeros_hbm, acc_sh.at[pl.ds(s * ROWS_PER_TILE, ROWS_PER_TILE)])
        plsc.subcore_barrier()

        nch = jnp.where(wid < (NCHUNKS % NTILES),
                        NCHUNKS // NTILES + 1, NCHUNKS // NTILES)

        def body(j, carry):
            off = (wid + j * NTILES) * CHUNK
            pltpu.sync_copy(src_hbm.at[pl.ds(off, CHUNK)], src_v)
            pltpu.sync_copy(dst_hbm.at[pl.ds(off, CHUNK)], dst_v)
            pltpu.async_copy(x_hbm.at[src_v], rows_v, sem).wait()
            pltpu.sync_copy(rows_v, acc_sh.at[dst_v], add=True)
            return carry

        lax.fori_loop(0, nch, body, 0)
        plsc.subcore_barrier()

        pltpu.sync_copy(acc_sh.at[pl.ds(s * ROWS_PER_TILE, ROWS_PER_TILE)],
                        out_hbm.at[pl.ds(c * NP + s * ROWS_PER_TILE, ROWS_PER_TILE)])

    return sc_agg


_sc_agg_aug = _make_sc_agg(W_AUG)
_sc_agg_d = _make_sc_agg(D)

_BLK = 1000


def _tc_layer1(x, p, w_self, w_neigh, b):
    def body(x_ref, p0_ref, p1_ref, ws_ref, wn_ref, b_ref, h_ref, inv_ref):
        agg = p0_ref[0, :, :D] + p1_ref[0, :, :D]
        deg = jnp.sum(p0_ref[0, :, D:] + p1_ref[0, :, D:], axis=1, keepdims=True)
        inv = 1.0 / jnp.maximum(deg, 1.0)
        hn = agg * inv
        h = (jnp.dot(x_ref[...], ws_ref[...], preferred_element_type=jnp.float32)
             + jnp.dot(hn, wn_ref[...], preferred_element_type=jnp.float32)
             + b_ref[...])
        h_ref[...] = jnp.maximum(h, 0.0)
        inv_ref[...] = jnp.broadcast_to(inv, (_BLK, 8))

    return pl.pallas_call(
        body,
        grid=(N // _BLK,),
        in_specs=[
            pl.BlockSpec((_BLK, D), lambda i: (i, 0)),
            pl.BlockSpec((1, _BLK, W_AUG), lambda i: (0, i, 0)),
            pl.BlockSpec((1, _BLK, W_AUG), lambda i: (1, i, 0)),
            pl.BlockSpec((D, D), lambda i: (0, 0)),
            pl.BlockSpec((D, D), lambda i: (0, 0)),
            pl.BlockSpec((1, D), lambda i: (0, 0)),
        ],
        out_specs=[
            pl.BlockSpec((_BLK, D), lambda i: (i, 0)),
            pl.BlockSpec((_BLK, 8), lambda i: (i, 0)),
        ],
        out_shape=[
            jax.ShapeDtypeStruct((N, D), jnp.float32),
            jax.ShapeDtypeStruct((N, 8), jnp.float32),
        ],
    )(x, p, p, w_self, w_neigh, b)


def _tc_layer2(h, q, inv, w_self, w_neigh, b):
    def body(h_ref, q0_ref, q1_ref, inv_ref, ws_ref, wn_ref, b_ref, o_ref):
        agg = q0_ref[0] + q1_ref[0]
        hn = agg * inv_ref[:, 0:1]
        o_ref[...] = (jnp.dot(h_ref[...], ws_ref[...], preferred_element_type=jnp.float32)
                      + jnp.dot(hn, wn_ref[...], preferred_element_type=jnp.float32)
                      + b_ref[...])

    return pl.pallas_call(
        body,
        grid=(N // _BLK,),
        in_specs=[
            pl.BlockSpec((_BLK, D), lambda i: (i, 0)),
            pl.BlockSpec((1, _BLK, D), lambda i: (0, i, 0)),
            pl.BlockSpec((1, _BLK, D), lambda i: (1, i, 0)),
            pl.BlockSpec((_BLK, 8), lambda i: (i, 0)),
            pl.BlockSpec((D, D), lambda i: (0, 0)),
            pl.BlockSpec((D, D), lambda i: (0, 0)),
            pl.BlockSpec((1, D), lambda i: (0, 0)),
        ],
        out_specs=pl.BlockSpec((_BLK, D), lambda i: (i, 0)),
        out_shape=jax.ShapeDtypeStruct((N, D), jnp.float32),
    )(h, q, q, inv, w_self, w_neigh, b)


def kernel(x, edge_index, W_self1, W_neigh1, b1, W_self2, W_neigh2, b2):
    src = edge_index[0].astype(jnp.int32)
    dst = edge_index[1].astype(jnp.int32)

    x_aug = jnp.concatenate(
        [x, jnp.ones((N, 1), jnp.float32), jnp.zeros((N, W_AUG - D - 1), jnp.float32)],
        axis=1)

    z_aug = jnp.zeros((ROWS_PER_TILE, W_AUG), jnp.float32)
    z_d = jnp.zeros((ROWS_PER_TILE, D), jnp.float32)

    p = _sc_agg_aug(x_aug, src, dst, z_aug).reshape(2, NP, W_AUG)
    h, inv = _tc_layer1(x, p, W_self1, W_neigh1, b1.reshape(1, D))
    q = _sc_agg_d(h, src, dst, z_d).reshape(2, NP, D)
    out = _tc_layer2(h, q, inv, W_self2, W_neigh2, b2.reshape(1, D))
    return out
